# baseline (speedup 1.0000x reference)
import jax
import jax.numpy as jnp
from jax import lax
from jax.experimental import pallas as pl
from jax.experimental.pallas import tpu as pltpu
from jax.experimental.pallas import tpu_sc as plsc
import functools

_WS = 16
_WT = 128


def _sc_body(x_hbm, rows_out, cnt_out, slab, cstore, loc2, rowbuf, shared, *, t):
    c = lax.axis_index("c")
    s = lax.axis_index("s")
    batch = c * 8 + s // 2
    half = s % 2
    ht = t // 2

    pltpu.sync_copy(
        x_hbm.at[batch, pl.ds(half * ht, ht), pl.ds(0, _WS)], slab
    )

    def blk(r8, cnt):
        for u in range(8):
            v = plsc.bitcast(slab[r8 * 8 + u, :], jnp.int32)
            nz = (v << 1) != 0
            pc = plsc.all_reduce_population_count(nz)
            cnt = cnt + (pc > 0).astype(jnp.int32)
        return cnt

    cnt = lax.fori_loop(0, ht // 8, blk, jnp.zeros((16,), jnp.int32))
    cstore[...] = cnt
    pltpu.sync_copy(cstore, shared.at[s])
    plsc.subcore_barrier()

    @pl.when(half == 0)
    def _():
        pltpu.sync_copy(shared.at[pl.ds(s, 2)], loc2)
        tot = loc2[0, :] + loc2[1, :]
        idx = lax.reduce_max(jnp.maximum(tot - 1, 0), axes=(0,))
        pltpu.sync_copy(x_hbm.at[batch, pl.ds(idx, 1), :], rowbuf)
        pltpu.sync_copy(rowbuf.at[0], rows_out.at[batch])
        cstore[...] = tot
        pltpu.sync_copy(cstore, cnt_out.at[batch])


def _sc_call(inputs):
    b, t, f = inputs.shape
    ht = t // 2
    mesh = plsc.VectorSubcoreMesh(core_axis_name="c", subcore_axis_name="s")
    rows, cnt = pl.kernel(
        functools.partial(_sc_body, t=t),
        out_type=[
            jax.ShapeDtypeStruct((b, f), jnp.float32),
            jax.ShapeDtypeStruct((b, 16), jnp.int32),
        ],
        mesh=mesh,
        scratch_types=[
            pltpu.VMEM((ht, _WS), jnp.float32),
            pltpu.VMEM((16,), jnp.int32),
            pltpu.VMEM((2, 16), jnp.int32),
            pltpu.VMEM((1, f), jnp.float32),
            pltpu.VMEM_SHARED((16, 16), jnp.int32),
        ],
        compiler_params=pltpu.CompilerParams(
            use_tc_tiling_on_sc=False, needs_layout_passes=False
        ),
    )(inputs)
    return rows, cnt[:, 0]


def _tc_body(x_hbm, o_ref, slab, fb, idx_smem, insem, fbsem, outsem):
    b, t, f = x_hbm.shape

    slab_copies = []
    for i in range(b):
        c = pltpu.make_async_copy(
            x_hbm.at[i, :, pl.ds(0, _WT)], slab.at[i], insem.at[i]
        )
        c.start()
        slab_copies.append(c)

    row_copies = []
    for i in range(b):
        slab_copies[i].wait()
        x = slab[i]
        m = jnp.max(jnp.abs(x), axis=1, keepdims=True)
        cnt = jnp.sum((m > 0.0).astype(jnp.int32))
        idx_smem[i] = jnp.maximum(cnt - 1, 0)

        @pl.when(cnt < t)
        def _():
            fc = pltpu.make_async_copy(x_hbm.at[i], fb, fbsem)
            fc.start()
            fc.wait()
            mf = jnp.max(jnp.abs(fb[...]), axis=1, keepdims=True)
            cf = jnp.sum((mf > 0.0).astype(jnp.int32))
            idx_smem[i] = jnp.maximum(cf - 1, 0)

        rc = pltpu.make_async_copy(
            x_hbm.at[i, pl.ds(idx_smem[i], 1), :],
            o_ref.at[pl.ds(i, 1)],
            outsem.at[i],
        )
        rc.start()
        row_copies.append(rc)

    for c in row_copies:
        c.wait()


def _tc_call(inputs):
    b, t, f = inputs.shape
    return pl.pallas_call(
        _tc_body,
        in_specs=[pl.BlockSpec(memory_space=pl.ANY)],
        out_specs=pl.BlockSpec((b, f), lambda: (0, 0)),
        out_shape=jax.ShapeDtypeStruct((b, f), jnp.float32),
        scratch_shapes=[
            pltpu.VMEM((b, t, _WT), jnp.float32),
            pltpu.VMEM((t, f), jnp.float32),
            pltpu.SMEM((b,), jnp.int32),
            pltpu.SemaphoreType.DMA((b,)),
            pltpu.SemaphoreType.DMA,
            pltpu.SemaphoreType.DMA((b,)),
        ],
    )(inputs)


def kernel(inputs):
    b, t, f = inputs.shape
    rows, cnt = _sc_call(inputs)
    return lax.cond(
        jnp.any(cnt < t),
        lambda: _tc_call(inputs),
        lambda: rows,
    )


# speedup vs baseline: 3.3218x; 3.3218x over previous
import jax
import jax.numpy as jnp
from jax import lax
from jax.experimental import pallas as pl
from jax.experimental.pallas import tpu as pltpu
from jax.experimental.pallas import tpu_sc as plsc
import functools

_W = 128


def _sc_body(x_hbm, rows_out, cnt_out, slab, cstore, loc4, rowbuf, shared, *, t):
    c = lax.axis_index("c")
    s = lax.axis_index("s")
    batch = 8 + c * 4 + s // 4
    quarter = s % 4
    qt = t // 4

    pltpu.sync_copy(
        x_hbm.at[batch, pl.ds(quarter * qt, qt), pl.ds(0, _W)], slab
    )

    def blk(r, cnt):
        acc = plsc.bitcast(slab[r, pl.ds(0, 16)], jnp.int32)
        for j in range(1, _W // 16):
            acc = acc | plsc.bitcast(slab[r, pl.ds(j * 16, 16)], jnp.int32)
        nz = (acc << 1) != 0
        pc = plsc.all_reduce_population_count(nz)
        return cnt + (pc > 0).astype(jnp.int32)

    cnt = lax.fori_loop(0, qt, blk, jnp.zeros((16,), jnp.int32))
    cstore[...] = cnt
    pltpu.sync_copy(cstore, shared.at[s])
    plsc.subcore_barrier()

    @pl.when(quarter == 0)
    def _():
        pltpu.sync_copy(shared.at[pl.ds(s, 4)], loc4)
        tot = loc4[0, :] + loc4[1, :] + loc4[2, :] + loc4[3, :]
        idx = lax.reduce_max(jnp.maximum(tot - 1, 0), axes=(0,))
        pltpu.sync_copy(x_hbm.at[batch, pl.ds(idx, 1), :], rowbuf)
        pltpu.sync_copy(rowbuf.at[0], rows_out.at[batch - 8])
        cstore[...] = tot
        pltpu.sync_copy(cstore, cnt_out.at[batch - 8])


def _sc_call(inputs):
    b, t, f = inputs.shape
    qt = t // 4
    mesh = plsc.VectorSubcoreMesh(core_axis_name="c", subcore_axis_name="s")
    rows, cnt = pl.kernel(
        functools.partial(_sc_body, t=t),
        out_type=[
            jax.ShapeDtypeStruct((8, f), jnp.float32),
            jax.ShapeDtypeStruct((8, 16), jnp.int32),
        ],
        mesh=mesh,
        scratch_types=[
            pltpu.VMEM((qt, _W), jnp.float32),
            pltpu.VMEM((16,), jnp.int32),
            pltpu.VMEM((4, 16), jnp.int32),
            pltpu.VMEM((1, f), jnp.float32),
            pltpu.VMEM_SHARED((16, 16), jnp.int32),
        ],
        compiler_params=pltpu.CompilerParams(needs_layout_passes=False),
    )(inputs)
    return rows, cnt[:, 0]


def _tc_body(x_hbm, o_ref, slab, fb, idx_smem, insem, fbsem, outsem, *, lo, hi):
    _, t, f = x_hbm.shape
    nb = hi - lo

    slab_copies = []
    for i in range(nb):
        c = pltpu.make_async_copy(
            x_hbm.at[lo + i, :, pl.ds(0, _W)], slab.at[i], insem.at[i]
        )
        c.start()
        slab_copies.append(c)

    row_copies = []
    for i in range(nb):
        slab_copies[i].wait()
        x = slab[i]
        m = jnp.max(jnp.abs(x), axis=1, keepdims=True)
        cnt = jnp.sum((m > 0.0).astype(jnp.int32))
        idx_smem[i] = jnp.maximum(cnt - 1, 0)

        @pl.when(cnt < t)
        def _():
            fc = pltpu.make_async_copy(x_hbm.at[lo + i], fb, fbsem)
            fc.start()
            fc.wait()
            mf = jnp.max(jnp.abs(fb[...]), axis=1, keepdims=True)
            cf = jnp.sum((mf > 0.0).astype(jnp.int32))
            idx_smem[i] = jnp.maximum(cf - 1, 0)

        rc = pltpu.make_async_copy(
            x_hbm.at[lo + i, pl.ds(idx_smem[i], 1), :],
            o_ref.at[pl.ds(i, 1)],
            outsem.at[i],
        )
        rc.start()
        row_copies.append(rc)

    for c in row_copies:
        c.wait()


def _tc_call(inputs, lo, hi):
    b, t, f = inputs.shape
    nb = hi - lo
    return pl.pallas_call(
        functools.partial(_tc_body, lo=lo, hi=hi),
        in_specs=[pl.BlockSpec(memory_space=pl.ANY)],
        out_specs=pl.BlockSpec((nb, f), lambda: (0, 0)),
        out_shape=jax.ShapeDtypeStruct((nb, f), jnp.float32),
        scratch_shapes=[
            pltpu.VMEM((nb, t, _W), jnp.float32),
            pltpu.VMEM((t, f), jnp.float32),
            pltpu.SMEM((nb,), jnp.int32),
            pltpu.SemaphoreType.DMA((nb,)),
            pltpu.SemaphoreType.DMA,
            pltpu.SemaphoreType.DMA((nb,)),
        ],
    )(inputs)


def kernel(inputs):
    b, t, f = inputs.shape
    rows_sc, cnt_sc = _sc_call(inputs)
    rows_tc = _tc_call(inputs, 0, 8)
    rows_hi = lax.cond(
        jnp.any(cnt_sc < t),
        lambda: _tc_call(inputs, 8, 16),
        lambda: rows_sc,
    )
    return jnp.concatenate([rows_tc, rows_hi], axis=0)


# speedup vs baseline: 13.2054x; 3.9753x over previous
import jax
import jax.numpy as jnp
from jax.experimental import pallas as pl
from jax.experimental.pallas import tpu as pltpu

_W = 128


def _body(x_hbm, o_ref, slab, fb, idx_smem, insem, fbsem, outsem):
    b, t, f = x_hbm.shape

    slab_copies = []
    for i in range(b):
        c = pltpu.make_async_copy(
            x_hbm.at[i, :, pl.ds(0, _W)], slab.at[i], insem.at[i]
        )
        c.start()
        slab_copies.append(c)

    row_copies = []
    for i in range(b):
        slab_copies[i].wait()
        x = slab[i]
        m = jnp.max(jnp.abs(x), axis=1, keepdims=True)
        cnt = jnp.sum((m > 0.0).astype(jnp.int32))
        idx_smem[i] = jnp.maximum(cnt - 1, 0)

        @pl.when(cnt < t)
        def _():
            fc = pltpu.make_async_copy(x_hbm.at[i], fb, fbsem)
            fc.start()
            fc.wait()
            mf = jnp.max(jnp.abs(fb[...]), axis=1, keepdims=True)
            cf = jnp.sum((mf > 0.0).astype(jnp.int32))
            idx_smem[i] = jnp.maximum(cf - 1, 0)

        rc = pltpu.make_async_copy(
            x_hbm.at[i, pl.ds(idx_smem[i], 1), :],
            o_ref.at[pl.ds(i, 1)],
            outsem.at[i],
        )
        rc.start()
        row_copies.append(rc)

    for c in row_copies:
        c.wait()


def kernel(inputs):
    b, t, f = inputs.shape

    return pl.pallas_call(
        _body,
        in_specs=[pl.BlockSpec(memory_space=pl.ANY)],
        out_specs=pl.BlockSpec((b, f), lambda: (0, 0)),
        out_shape=jax.ShapeDtypeStruct((b, f), jnp.float32),
        scratch_shapes=[
            pltpu.VMEM((b, t, _W), jnp.float32),
            pltpu.VMEM((t, f), jnp.float32),
            pltpu.SMEM((b,), jnp.int32),
            pltpu.SemaphoreType.DMA((b,)),
            pltpu.SemaphoreType.DMA,
            pltpu.SemaphoreType.DMA((b,)),
        ],
    )(inputs)
